# baseline (speedup 1.0000x reference)
import functools

import jax
import jax.numpy as jnp
from jax import lax
from jax.experimental import pallas as pl
from jax.experimental.pallas import tpu as pltpu
from jax.experimental.pallas import tpu_sc as plsc

F32 = jnp.float32


def _node_body(at_ref, rf_ref, pf_ref, bt_ref, aemb_ref, afW_ref, t_ref,
               ztWh1_ref, ztWh2_ref, wtsum_ref, ztb_ref,
               hW0_ref, hWr_ref, hWp_ref,
               W1a_ref, W1b_ref, b1_ref, ns_ref, nd_ref):
    B = at_ref.shape[0]
    NAT = aemb_ref.shape[0]
    NG = t_ref.shape[0]
    at = at_ref[...]
    aoh = (at == lax.broadcasted_iota(jnp.int32, (B, NAT), 1)).astype(F32)
    ae = jnp.dot(aoh, aemb_ref[...], preferred_element_type=F32)

    def onehot80(f_ref):
        cols = []
        for f in range(8):
            oh = (f_ref[:, f:f + 1] ==
                  lax.broadcasted_iota(jnp.int32, (B, 10), 1)).astype(F32)
            cols.append(oh)
        return jnp.concatenate(cols, axis=1)

    roh = onehot80(rf_ref)
    poh = onehot80(pf_ref)
    h1 = ae * jnp.dot(roh, afW_ref[...], preferred_element_type=F32)
    h2 = ae * jnp.dot(poh, afW_ref[...], preferred_element_type=F32)

    boh = (bt_ref[...] == lax.broadcasted_iota(jnp.int32, (B, NG), 1)).astype(F32)
    tn = jnp.dot(boh, t_ref[...], preferred_element_type=F32)

    node = (jnp.dot(h1, ztWh1_ref[...], preferred_element_type=F32)
            + jnp.dot(h2, ztWh2_ref[...], preferred_element_type=F32)
            + jnp.dot(tn, wtsum_ref[...], preferred_element_type=F32)
            + jnp.dot(at.astype(F32), hW0_ref[...], preferred_element_type=F32)
            + jnp.dot(roh, hWr_ref[...], preferred_element_type=F32)
            + jnp.dot(poh, hWp_ref[...], preferred_element_type=F32)
            + ztb_ref[...])
    ns_ref[...] = jnp.dot(node, W1a_ref[...], preferred_element_type=F32) + b1_ref[...]
    nd_ref[...] = jnp.dot(node, W1b_ref[...], preferred_element_type=F32)


def _edge_body(gsum_ref, gps_ref, gpd_ref, etr_ref, etp_ref,
               eW1a_ref, eW1b_ref, eb1_ref, eW2_ref, eb2_ref, bemb_ref,
               Ct_ref, Cb_ref, wd_ref, W2_ref, b2_ref, cW_ref, out_ref):
    B = gsum_ref.shape[0]
    NBT = bemb_ref.shape[0]
    dv = gps_ref[...] - gpd_ref[...]
    d = jnp.sqrt(jnp.sum(dv[:, 0:3] * dv[:, 0:3], axis=1, keepdims=True) + 1e-9)
    dT = jnp.sqrt(jnp.sum(dv[:, 3:6] * dv[:, 3:6], axis=1, keepdims=True) + 1e-9)
    e1 = jnp.maximum(d * eW1a_ref[...] + dT * eW1b_ref[...] + eb1_ref[...], 0.0)
    e_mlp = jnp.dot(e1, eW2_ref[...], preferred_element_type=F32) + eb2_ref[...]
    br = jnp.dot((etr_ref[...] == lax.broadcasted_iota(jnp.int32, (B, NBT), 1)
                  ).astype(F32), bemb_ref[...], preferred_element_type=F32)
    bp = jnp.dot((etp_ref[...] == lax.broadcasted_iota(jnp.int32, (B, NBT), 1)
                  ).astype(F32), bemb_ref[...], preferred_element_type=F32)
    pre = (gsum_ref[...]
           + jnp.dot(e_mlp * br, Ct_ref[...], preferred_element_type=F32)
           + jnp.dot(e_mlp * bp, Cb_ref[...], preferred_element_type=F32)
           + d * wd_ref[...])
    m = jnp.dot(jnp.maximum(pre, 0.0), W2_ref[...], preferred_element_type=F32) + b2_ref[...]
    phi = jnp.dot(m, cW_ref[...], preferred_element_type=F32)
    out = dv[:, 0:3] * phi
    out_ref[...] = jnp.concatenate([out, jnp.zeros((B, 5), F32)], axis=1)


def _fin_body(pos_ref, a0_ref, a1_ref, a2_ref, a3_ref, out_ref):
    out_ref[...] = (pos_ref[...] + (a0_ref[...] + a1_ref[...])
                    + (a2_ref[...] + a3_ref[...]))


def kernel(atom_type, r_feat, p_feat, pos, pos_init, edge_index, edge_type_r,
           edge_type_p, t, batch, atom_emb, atom_feat_W, bond_emb, edge_W1,
           edge_b1, edge_W2, edge_b2, zt_W, zt_b, h_W, msg_W1, msg_b1,
           msg_W2, msg_b2, coord_W):
    N = atom_type.shape[0]
    E = edge_index.shape[1]
    HD = msg_W2.shape[0]
    HH = HD // 2

    at2 = atom_type[:, None].astype(jnp.int32)
    bt2 = batch[:, None].astype(jnp.int32)
    t2 = t[:, None]
    ztWh1 = zt_W[0:HH]
    ztWh2 = zt_W[HH:HD]
    hW0 = h_W[0:1]
    hWr = h_W[1:81]
    hWp = h_W[81:161]
    wtsum = zt_W[HD:HD + 1] + h_W[161:162]
    W1a = msg_W1[0:HD]
    W1b = msg_W1[HD:2 * HD]
    Ct = msg_W1[2 * HD:2 * HD + HH]
    Cb = msg_W1[2 * HD + HH:3 * HD]
    wd = msg_W1[3 * HD:3 * HD + 1]
    row1 = lambda v: v[None, :]

    NB = 1000
    n_blocks = N // NB
    full = lambda shp: pl.BlockSpec(shp, lambda i: (0, 0))
    ns, nd = pl.pallas_call(
        _node_body,
        grid=(n_blocks,),
        in_specs=[
            pl.BlockSpec((NB, 1), lambda i: (i, 0)),
            pl.BlockSpec((NB, 8), lambda i: (i, 0)),
            pl.BlockSpec((NB, 8), lambda i: (i, 0)),
            pl.BlockSpec((NB, 1), lambda i: (i, 0)),
            full(atom_emb.shape), full(atom_feat_W.shape), full(t2.shape),
            full(ztWh1.shape), full(ztWh2.shape), full(wtsum.shape),
            full((1, HD)),
            full(hW0.shape), full(hWr.shape), full(hWp.shape),
            full(W1a.shape), full(W1b.shape), full((1, HD)),
        ],
        out_specs=[pl.BlockSpec((NB, HD), lambda i: (i, 0)),
                   pl.BlockSpec((NB, HD), lambda i: (i, 0))],
        out_shape=[jax.ShapeDtypeStruct((N, HD), F32),
                   jax.ShapeDtypeStruct((N, HD), F32)],
    )(at2, r_feat.astype(jnp.int32), p_feat.astype(jnp.int32), bt2,
      atom_emb, atom_feat_W, t2, ztWh1, ztWh2, wtsum, row1(zt_b),
      hW0, hWr, hWp, W1a, W1b, row1(msg_b1))

    posc = jnp.concatenate([pos, pos_init, jnp.zeros((N, 2), F32)], axis=1)
    src = edge_index[0].astype(jnp.int32)
    dst = edge_index[1].astype(jnp.int32)

    info = plsc.get_sparse_core_info()
    NWC, NWS = info.num_cores, info.num_subcores
    NW = NWC * NWS
    ES = E // 2
    EPW = ES // NW
    CH = 40
    NCH = EPW // CH

    mesh = plsc.VectorSubcoreMesh(core_axis_name="c", subcore_axis_name="s")

    sc_params = pltpu.CompilerParams(use_tc_tiling_on_sc=False)

    @functools.partial(
        pl.kernel, mesh=mesh, compiler_params=sc_params,
        out_type=[jax.ShapeDtypeStruct((ES, HD), F32),
                  jax.ShapeDtypeStruct((ES, 8), F32),
                  jax.ShapeDtypeStruct((ES, 8), F32)],
        scratch_types=[
            pltpu.VMEM((EPW,), jnp.int32), pltpu.VMEM((EPW,), jnp.int32),
            pltpu.VMEM((CH, HD), F32), pltpu.VMEM((CH, HD), F32),
            pltpu.VMEM((CH, HD), F32), pltpu.VMEM((CH, HD), F32),
            pltpu.VMEM((CH, 8), F32), pltpu.VMEM((CH, 8), F32),
            pltpu.VMEM((CH, 8), F32), pltpu.VMEM((CH, 8), F32),
            pltpu.SemaphoreType.DMA, pltpu.SemaphoreType.DMA,
            pltpu.SemaphoreType.DMA, pltpu.SemaphoreType.DMA,
        ],
    )
    def _gather_k(src_h, dst_h, ns_h, nd_h, posc_h,
                  gs_h, ps_h, pd_h,
                  idxs_v, idxd_v, rs0, rs1, rd0, rd1, pvs0, pvs1, pvd0, pvd1,
                  gsem0, gsem1, wsem0, wsem1):
        wid = lax.axis_index("s") * NWC + lax.axis_index("c")
        base = wid * EPW
        bufs = ((rs0, rd0, pvs0, pvd0), (rs1, rd1, pvs1, pvd1))
        gsems = (gsem0, gsem1)
        wsems = (wsem0, wsem1)
        pltpu.sync_copy(src_h.at[pl.ds(base, EPW)], idxs_v)
        pltpu.sync_copy(dst_h.at[pl.ds(base, EPW)], idxd_v)

        def fire_gathers(g, b):
            rs, rd, pvs, pvd = bufs[b]
            isl = idxs_v.at[pl.ds(g * CH, CH)]
            dsl = idxd_v.at[pl.ds(g * CH, CH)]
            pltpu.async_copy(ns_h.at[isl], rs, gsems[b])
            pltpu.async_copy(nd_h.at[dsl], rd, gsems[b])
            pltpu.async_copy(posc_h.at[isl], pvs, gsems[b])
            pltpu.async_copy(posc_h.at[dsl], pvd, gsems[b])

        def drain_gathers(b):
            rs, rd, pvs, pvd = bufs[b]
            pltpu.make_async_copy(ns_h.at[pl.ds(0, CH)], rs, gsems[b]).wait()
            pltpu.make_async_copy(nd_h.at[pl.ds(0, CH)], rd, gsems[b]).wait()
            pltpu.make_async_copy(posc_h.at[pl.ds(0, CH)], pvs, gsems[b]).wait()
            pltpu.make_async_copy(posc_h.at[pl.ds(0, CH)], pvd, gsems[b]).wait()

        def sum_rows(b):
            rs, rd, _, _ = bufs[b]

            def add_row(r, carry):
                for c in range(HD // 16):
                    sl = pl.ds(c * 16, 16)
                    rs[r, sl] = rs[r, sl] + rd[r, sl]
                return carry

            lax.fori_loop(0, CH, add_row, 0)

        def fire_writes(g, b):
            rs, rd, pvs, pvd = bufs[b]
            off = base + g * CH
            pltpu.async_copy(rs, gs_h.at[pl.ds(off, CH)], wsems[b])
            pltpu.async_copy(pvs, ps_h.at[pl.ds(off, CH)], wsems[b])
            pltpu.async_copy(pvd, pd_h.at[pl.ds(off, CH)], wsems[b])

        def drain_writes(b):
            rs, rd, pvs, pvd = bufs[b]
            pltpu.make_async_copy(rs, gs_h.at[pl.ds(0, CH)], wsems[b]).wait()
            pltpu.make_async_copy(pvs, ps_h.at[pl.ds(0, CH)], wsems[b]).wait()
            pltpu.make_async_copy(pvd, pd_h.at[pl.ds(0, CH)], wsems[b]).wait()

        fire_gathers(0, 0)

        def body(g, carry):
            @pl.when(g % 2 == 0)
            def _():
                @pl.when(g + 1 < NCH)
                def _():
                    @pl.when(g >= 1)
                    def _():
                        drain_writes(1)
                    fire_gathers(g + 1, 1)
                drain_gathers(0)
                sum_rows(0)
                fire_writes(g, 0)

            @pl.when(g % 2 == 1)
            def _():
                @pl.when(g + 1 < NCH)
                def _():
                    drain_writes(0)
                    fire_gathers(g + 1, 0)
                drain_gathers(1)
                sum_rows(1)
                fire_writes(g, 1)

            return carry

        lax.fori_loop(0, NCH, body, 0)
        drain_writes((NCH - 2) % 2)
        drain_writes((NCH - 1) % 2)

    halves = []
    for h in range(2):
        sl = slice(h * ES, (h + 1) * ES)
        halves.append((_gather_k(src[sl], dst[sl], ns, nd, posc), dst[sl]))

    EB = 3200
    e_blocks = ES // EB

    def _run_edge(h, gsum, gps, gpd):
        sl = slice(h * ES, (h + 1) * ES)
        return pl.pallas_call(
            _edge_body,
            grid=(e_blocks,),
            in_specs=[
                pl.BlockSpec((EB, HD), lambda i: (i, 0)),
                pl.BlockSpec((EB, 8), lambda i: (i, 0)),
                pl.BlockSpec((EB, 8), lambda i: (i, 0)),
                pl.BlockSpec((EB, 1), lambda i: (i, 0)),
                pl.BlockSpec((EB, 1), lambda i: (i, 0)),
                full((1, HH)), full((1, HH)), full((1, HH)), full(edge_W2.shape),
                full((1, HH)), full(bond_emb.shape),
                full(Ct.shape), full(Cb.shape), full(wd.shape),
                full(msg_W2.shape), full((1, HD)), full(coord_W.shape),
            ],
            out_specs=pl.BlockSpec((EB, 8), lambda i: (i, 0)),
            out_shape=jax.ShapeDtypeStruct((ES, 8), F32),
        )(gsum, gps, gpd, edge_type_r[sl, None].astype(jnp.int32),
          edge_type_p[sl, None].astype(jnp.int32),
          edge_W1[0:1], edge_W1[1:2], row1(edge_b1), edge_W2, row1(edge_b2),
          bond_emb, Ct, Cb, wd, msg_W2, row1(msg_b2), coord_W)

    edgeouts = [_run_edge(h, *halves[h][0]) for h in range(2)]

    NPAD = 10240
    RPT = NPAD // NWS
    zeros_acc = jnp.zeros((NPAD, 8), F32)

    @functools.partial(
        pl.kernel, mesh=mesh, compiler_params=sc_params,
        out_type=[jax.ShapeDtypeStruct((2 * NPAD, 8), F32)],
        scratch_types=[
            pltpu.VMEM((CH,), jnp.int32),
            pltpu.VMEM((CH, 8), F32),
            pltpu.VMEM((RPT, 8), F32),
            pltpu.VMEM_SHARED((NPAD, 8), F32),
        ],
    )
    def _scatter_k(dst_h, eo_h, z_h, agg_h, idx_v, rows_v, tmp_v, acc_s):
        cid = lax.axis_index("c")
        sid = lax.axis_index("s")
        wid = sid * NWC + cid
        base = wid * EPW
        pltpu.sync_copy(z_h.at[pl.ds(sid * RPT, RPT)], tmp_v)
        pltpu.sync_copy(tmp_v, acc_s.at[pl.ds(sid * RPT, RPT)])
        plsc.subcore_barrier()

        def body(g, carry):
            off = base + g * CH
            pltpu.sync_copy(dst_h.at[pl.ds(off, CH)], idx_v)
            pltpu.sync_copy(eo_h.at[pl.ds(off, CH)], rows_v)
            pltpu.sync_copy(rows_v, acc_s.at[idx_v], add=True)
            return carry

        lax.fori_loop(0, NCH, body, 0)
        plsc.subcore_barrier()
        pltpu.sync_copy(acc_s.at[pl.ds(sid * RPT, RPT)], tmp_v)
        pltpu.sync_copy(tmp_v, agg_h.at[pl.ds(cid * NPAD + sid * RPT, RPT)])

    parts = []
    for h in range(2):
        (agg,) = _scatter_k(halves[h][1], edgeouts[h], zeros_acc)
        parts.append(agg[0:N, 0:3])
        parts.append(agg[NPAD:NPAD + N, 0:3])

    FB = 1000
    pred = pl.pallas_call(
        _fin_body,
        grid=(N // FB,),
        in_specs=[pl.BlockSpec((FB, 3), lambda i: (i, 0))] * 5,
        out_specs=pl.BlockSpec((FB, 3), lambda i: (i, 0)),
        out_shape=jax.ShapeDtypeStruct((N, 3), F32),
    )(pos, *parts)
    return pred


# speedup vs baseline: 1.0190x; 1.0190x over previous
import functools

import jax
import jax.numpy as jnp
from jax import lax
from jax.experimental import pallas as pl
from jax.experimental.pallas import tpu as pltpu
from jax.experimental.pallas import tpu_sc as plsc

F32 = jnp.float32


def _node_body(at_ref, rf_ref, pf_ref, bt_ref, aemb_ref, afW_ref, t_ref,
               ztWh1_ref, ztWh2_ref, wtsum_ref, ztb_ref,
               hW0_ref, hWr_ref, hWp_ref,
               W1a_ref, W1b_ref, b1_ref, ns_ref, nd_ref):
    B = at_ref.shape[0]
    NAT = aemb_ref.shape[0]
    NG = t_ref.shape[0]
    at = at_ref[...]
    aoh = (at == lax.broadcasted_iota(jnp.int32, (B, NAT), 1)).astype(F32)
    ae = jnp.dot(aoh, aemb_ref[...], preferred_element_type=F32)

    def onehot80(f_ref):
        cols = []
        for f in range(8):
            oh = (f_ref[:, f:f + 1] ==
                  lax.broadcasted_iota(jnp.int32, (B, 10), 1)).astype(F32)
            cols.append(oh)
        return jnp.concatenate(cols, axis=1)

    roh = onehot80(rf_ref)
    poh = onehot80(pf_ref)
    h1 = ae * jnp.dot(roh, afW_ref[...], preferred_element_type=F32)
    h2 = ae * jnp.dot(poh, afW_ref[...], preferred_element_type=F32)

    boh = (bt_ref[...] == lax.broadcasted_iota(jnp.int32, (B, NG), 1)).astype(F32)
    tn = jnp.dot(boh, t_ref[...], preferred_element_type=F32)

    node = (jnp.dot(h1, ztWh1_ref[...], preferred_element_type=F32)
            + jnp.dot(h2, ztWh2_ref[...], preferred_element_type=F32)
            + jnp.dot(tn, wtsum_ref[...], preferred_element_type=F32)
            + jnp.dot(at.astype(F32), hW0_ref[...], preferred_element_type=F32)
            + jnp.dot(roh, hWr_ref[...], preferred_element_type=F32)
            + jnp.dot(poh, hWp_ref[...], preferred_element_type=F32)
            + ztb_ref[...])
    ns_ref[...] = jnp.dot(node, W1a_ref[...], preferred_element_type=F32) + b1_ref[...]
    nd_ref[...] = jnp.dot(node, W1b_ref[...], preferred_element_type=F32)


def _edge_body(gsum_ref, gps_ref, gpd_ref, etr_ref, etp_ref,
               eW1a_ref, eW1b_ref, eb1_ref, eW2_ref, eb2_ref, bemb_ref,
               Ct_ref, Cb_ref, wd_ref, W2_ref, b2_ref, cW_ref, out_ref):
    B = gsum_ref.shape[0]
    NBT = bemb_ref.shape[0]
    dv = gps_ref[...] - gpd_ref[...]
    d = jnp.sqrt(jnp.sum(dv[:, 0:3] * dv[:, 0:3], axis=1, keepdims=True) + 1e-9)
    dT = jnp.sqrt(jnp.sum(dv[:, 3:6] * dv[:, 3:6], axis=1, keepdims=True) + 1e-9)
    e1 = jnp.maximum(d * eW1a_ref[...] + dT * eW1b_ref[...] + eb1_ref[...], 0.0)
    e_mlp = jnp.dot(e1, eW2_ref[...], preferred_element_type=F32) + eb2_ref[...]
    br = jnp.dot((etr_ref[...] == lax.broadcasted_iota(jnp.int32, (B, NBT), 1)
                  ).astype(F32), bemb_ref[...], preferred_element_type=F32)
    bp = jnp.dot((etp_ref[...] == lax.broadcasted_iota(jnp.int32, (B, NBT), 1)
                  ).astype(F32), bemb_ref[...], preferred_element_type=F32)
    pre = (gsum_ref[...]
           + jnp.dot(e_mlp * br, Ct_ref[...], preferred_element_type=F32)
           + jnp.dot(e_mlp * bp, Cb_ref[...], preferred_element_type=F32)
           + d * wd_ref[...])
    m = jnp.dot(jnp.maximum(pre, 0.0), W2_ref[...], preferred_element_type=F32) + b2_ref[...]
    phi = jnp.dot(m, cW_ref[...], preferred_element_type=F32)
    out = dv[:, 0:3] * phi
    out_ref[...] = jnp.concatenate([out, jnp.zeros((B, 5), F32)], axis=1)


def _fin_body(pos_ref, a0_ref, a1_ref, out_ref):
    out_ref[...] = pos_ref[...] + a0_ref[...] + a1_ref[...]


def kernel(atom_type, r_feat, p_feat, pos, pos_init, edge_index, edge_type_r,
           edge_type_p, t, batch, atom_emb, atom_feat_W, bond_emb, edge_W1,
           edge_b1, edge_W2, edge_b2, zt_W, zt_b, h_W, msg_W1, msg_b1,
           msg_W2, msg_b2, coord_W):
    N = atom_type.shape[0]
    E = edge_index.shape[1]
    HD = msg_W2.shape[0]
    HH = HD // 2

    at2 = atom_type[:, None].astype(jnp.int32)
    bt2 = batch[:, None].astype(jnp.int32)
    t2 = t[:, None]
    ztWh1 = zt_W[0:HH]
    ztWh2 = zt_W[HH:HD]
    hW0 = h_W[0:1]
    hWr = h_W[1:81]
    hWp = h_W[81:161]
    wtsum = zt_W[HD:HD + 1] + h_W[161:162]
    W1a = msg_W1[0:HD]
    W1b = msg_W1[HD:2 * HD]
    Ct = msg_W1[2 * HD:2 * HD + HH]
    Cb = msg_W1[2 * HD + HH:3 * HD]
    wd = msg_W1[3 * HD:3 * HD + 1]
    row1 = lambda v: v[None, :]

    NB = 1000
    n_blocks = N // NB
    full = lambda shp: pl.BlockSpec(shp, lambda i: (0, 0))
    ns, nd = pl.pallas_call(
        _node_body,
        grid=(n_blocks,),
        in_specs=[
            pl.BlockSpec((NB, 1), lambda i: (i, 0)),
            pl.BlockSpec((NB, 8), lambda i: (i, 0)),
            pl.BlockSpec((NB, 8), lambda i: (i, 0)),
            pl.BlockSpec((NB, 1), lambda i: (i, 0)),
            full(atom_emb.shape), full(atom_feat_W.shape), full(t2.shape),
            full(ztWh1.shape), full(ztWh2.shape), full(wtsum.shape),
            full((1, HD)),
            full(hW0.shape), full(hWr.shape), full(hWp.shape),
            full(W1a.shape), full(W1b.shape), full((1, HD)),
        ],
        out_specs=[pl.BlockSpec((NB, HD), lambda i: (i, 0)),
                   pl.BlockSpec((NB, HD), lambda i: (i, 0))],
        out_shape=[jax.ShapeDtypeStruct((N, HD), F32),
                   jax.ShapeDtypeStruct((N, HD), F32)],
    )(at2, r_feat.astype(jnp.int32), p_feat.astype(jnp.int32), bt2,
      atom_emb, atom_feat_W, t2, ztWh1, ztWh2, wtsum, row1(zt_b),
      hW0, hWr, hWp, W1a, W1b, row1(msg_b1))

    posc = jnp.concatenate([pos, pos_init, jnp.zeros((N, 2), F32)], axis=1)
    src = edge_index[0].astype(jnp.int32)
    dst = edge_index[1].astype(jnp.int32)

    info = plsc.get_sparse_core_info()
    NWC, NWS = info.num_cores, info.num_subcores
    NW = NWC * NWS
    EPW = E // NW
    CH = 80
    NCH = EPW // CH

    mesh = plsc.VectorSubcoreMesh(core_axis_name="c", subcore_axis_name="s")

    sc_params = pltpu.CompilerParams(use_tc_tiling_on_sc=False)

    @functools.partial(
        pl.kernel, mesh=mesh, compiler_params=sc_params,
        out_type=[jax.ShapeDtypeStruct((E, HD), F32),
                  jax.ShapeDtypeStruct((E, 8), F32),
                  jax.ShapeDtypeStruct((E, 8), F32)],
        scratch_types=[
            pltpu.VMEM((EPW,), jnp.int32), pltpu.VMEM((EPW,), jnp.int32),
            pltpu.VMEM((CH, HD), F32), pltpu.VMEM((CH, HD), F32),
            pltpu.VMEM((CH, HD), F32),
            pltpu.VMEM((CH, HD), F32), pltpu.VMEM((CH, HD), F32),
            pltpu.VMEM((CH, HD), F32),
            pltpu.VMEM((CH, 8), F32), pltpu.VMEM((CH, 8), F32),
            pltpu.VMEM((CH, 8), F32),
            pltpu.VMEM((CH, 8), F32), pltpu.VMEM((CH, 8), F32),
            pltpu.VMEM((CH, 8), F32),
            pltpu.SemaphoreType.DMA, pltpu.SemaphoreType.DMA,
            pltpu.SemaphoreType.DMA, pltpu.SemaphoreType.DMA,
            pltpu.SemaphoreType.DMA, pltpu.SemaphoreType.DMA,
        ],
    )
    def _gather_k(src_h, dst_h, ns_h, nd_h, posc_h,
                  gs_h, ps_h, pd_h,
                  idxs_v, idxd_v, rs0, rs1, rs2, rd0, rd1, rd2,
                  pvs0, pvs1, pvs2, pvd0, pvd1, pvd2,
                  gsem0, gsem1, gsem2, wsem0, wsem1, wsem2):
        wid = lax.axis_index("s") * NWC + lax.axis_index("c")
        base = wid * EPW
        bufs = ((rs0, rd0, pvs0, pvd0), (rs1, rd1, pvs1, pvd1),
                (rs2, rd2, pvs2, pvd2))
        gsems = (gsem0, gsem1, gsem2)
        wsems = (wsem0, wsem1, wsem2)
        pltpu.sync_copy(src_h.at[pl.ds(base, EPW)], idxs_v)
        pltpu.sync_copy(dst_h.at[pl.ds(base, EPW)], idxd_v)

        def fire_gathers(g, b):
            rs, rd, pvs, pvd = bufs[b]
            isl = idxs_v.at[pl.ds(g * CH, CH)]
            dsl = idxd_v.at[pl.ds(g * CH, CH)]
            pltpu.async_copy(ns_h.at[isl], rs, gsems[b])
            pltpu.async_copy(nd_h.at[dsl], rd, gsems[b])
            pltpu.async_copy(posc_h.at[isl], pvs, gsems[b])
            pltpu.async_copy(posc_h.at[dsl], pvd, gsems[b])

        def drain_gathers(b):
            rs, rd, pvs, pvd = bufs[b]
            pltpu.make_async_copy(ns_h.at[pl.ds(0, CH)], rs, gsems[b]).wait()
            pltpu.make_async_copy(nd_h.at[pl.ds(0, CH)], rd, gsems[b]).wait()
            pltpu.make_async_copy(posc_h.at[pl.ds(0, CH)], pvs, gsems[b]).wait()
            pltpu.make_async_copy(posc_h.at[pl.ds(0, CH)], pvd, gsems[b]).wait()

        def sum_rows(b):
            rs, rd, _, _ = bufs[b]

            def add_row(r, carry):
                for c in range(HD // 16):
                    sl = pl.ds(c * 16, 16)
                    rs[r, sl] = rs[r, sl] + rd[r, sl]
                return carry

            lax.fori_loop(0, CH, add_row, 0)

        def fire_writes(g, b):
            rs, rd, pvs, pvd = bufs[b]
            off = base + g * CH
            pltpu.async_copy(rs, gs_h.at[pl.ds(off, CH)], wsems[b])
            pltpu.async_copy(pvs, ps_h.at[pl.ds(off, CH)], wsems[b])
            pltpu.async_copy(pvd, pd_h.at[pl.ds(off, CH)], wsems[b])

        def drain_writes(b):
            rs, rd, pvs, pvd = bufs[b]
            pltpu.make_async_copy(rs, gs_h.at[pl.ds(0, CH)], wsems[b]).wait()
            pltpu.make_async_copy(pvs, ps_h.at[pl.ds(0, CH)], wsems[b]).wait()
            pltpu.make_async_copy(pvd, pd_h.at[pl.ds(0, CH)], wsems[b]).wait()

        fire_gathers(0, 0)
        fire_gathers(1, 1)

        def step(g, b):
            nb = (b + 2) % 3

            @pl.when(g + 2 < NCH)
            def _():
                @pl.when(g >= 1)
                def _():
                    drain_writes(nb)
                fire_gathers(g + 2, nb)
            drain_gathers(b)
            sum_rows(b)
            fire_writes(g, b)

        def body(g, carry):
            @pl.when(g % 3 == 0)
            def _():
                step(g, 0)

            @pl.when(g % 3 == 1)
            def _():
                step(g, 1)

            @pl.when(g % 3 == 2)
            def _():
                step(g, 2)

            return carry

        lax.fori_loop(0, NCH, body, 0)
        drain_writes((NCH - 3) % 3)
        drain_writes((NCH - 2) % 3)
        drain_writes((NCH - 1) % 3)

    gsum, gps, gpd = _gather_k(src, dst, ns, nd, posc)

    EB = 3200
    e_blocks = E // EB
    edgeout = pl.pallas_call(
        _edge_body,
        grid=(e_blocks,),
        in_specs=[
            pl.BlockSpec((EB, HD), lambda i: (i, 0)),
            pl.BlockSpec((EB, 8), lambda i: (i, 0)),
            pl.BlockSpec((EB, 8), lambda i: (i, 0)),
            pl.BlockSpec((EB, 1), lambda i: (i, 0)),
            pl.BlockSpec((EB, 1), lambda i: (i, 0)),
            full((1, HH)), full((1, HH)), full((1, HH)), full(edge_W2.shape),
            full((1, HH)), full(bond_emb.shape),
            full(Ct.shape), full(Cb.shape), full(wd.shape),
            full(msg_W2.shape), full((1, HD)), full(coord_W.shape),
        ],
        out_specs=pl.BlockSpec((EB, 8), lambda i: (i, 0)),
        out_shape=jax.ShapeDtypeStruct((E, 8), F32),
    )(gsum, gps, gpd, edge_type_r[:, None].astype(jnp.int32),
      edge_type_p[:, None].astype(jnp.int32),
      edge_W1[0:1], edge_W1[1:2], row1(edge_b1), edge_W2, row1(edge_b2),
      bond_emb, Ct, Cb, wd, msg_W2, row1(msg_b2), coord_W)

    NPAD = 10240
    RPT = NPAD // NWS
    zeros_acc = jnp.zeros((NPAD, 8), F32)

    @functools.partial(
        pl.kernel, mesh=mesh, compiler_params=sc_params,
        out_type=[jax.ShapeDtypeStruct((2 * NPAD, 8), F32)],
        scratch_types=[
            pltpu.VMEM((CH,), jnp.int32),
            pltpu.VMEM((CH, 8), F32),
            pltpu.VMEM((RPT, 8), F32),
            pltpu.VMEM_SHARED((NPAD, 8), F32),
        ],
    )
    def _scatter_k(dst_h, eo_h, z_h, agg_h, idx_v, rows_v, tmp_v, acc_s):
        cid = lax.axis_index("c")
        sid = lax.axis_index("s")
        wid = sid * NWC + cid
        base = wid * EPW
        pltpu.sync_copy(z_h.at[pl.ds(sid * RPT, RPT)], tmp_v)
        pltpu.sync_copy(tmp_v, acc_s.at[pl.ds(sid * RPT, RPT)])
        plsc.subcore_barrier()

        def body(g, carry):
            off = base + g * CH
            pltpu.sync_copy(dst_h.at[pl.ds(off, CH)], idx_v)
            pltpu.sync_copy(eo_h.at[pl.ds(off, CH)], rows_v)
            pltpu.sync_copy(rows_v, acc_s.at[idx_v], add=True)
            return carry

        lax.fori_loop(0, NCH, body, 0)
        plsc.subcore_barrier()
        pltpu.sync_copy(acc_s.at[pl.ds(sid * RPT, RPT)], tmp_v)
        pltpu.sync_copy(tmp_v, agg_h.at[pl.ds(cid * NPAD + sid * RPT, RPT)])

    (agg,) = _scatter_k(dst, edgeout, zeros_acc)

    a0 = agg[0:N, 0:3]
    a1 = agg[NPAD:NPAD + N, 0:3]
    FB = 1000
    pred = pl.pallas_call(
        _fin_body,
        grid=(N // FB,),
        in_specs=[pl.BlockSpec((FB, 3), lambda i: (i, 0))] * 3,
        out_specs=pl.BlockSpec((FB, 3), lambda i: (i, 0)),
        out_shape=jax.ShapeDtypeStruct((N, 3), F32),
    )(pos, a0, a1)
    return pred


# speedup vs baseline: 1.0415x; 1.0221x over previous
import functools

import jax
import jax.numpy as jnp
from jax import lax
from jax.experimental import pallas as pl
from jax.experimental.pallas import tpu as pltpu
from jax.experimental.pallas import tpu_sc as plsc

F32 = jnp.float32


def _node_body(at_ref, rf_ref, pf_ref, bt_ref, aemb_ref, afW_ref, t_ref,
               ztWh1_ref, ztWh2_ref, wtsum_ref, ztb_ref,
               hW0_ref, hWr_ref, hWp_ref,
               W1a_ref, W1b_ref, b1_ref, ns_ref, nd_ref):
    B = at_ref.shape[0]
    NAT = aemb_ref.shape[0]
    NG = t_ref.shape[0]
    at = at_ref[...]
    aoh = (at == lax.broadcasted_iota(jnp.int32, (B, NAT), 1)).astype(F32)
    ae = jnp.dot(aoh, aemb_ref[...], preferred_element_type=F32)

    def onehot80(f_ref):
        cols = []
        for f in range(8):
            oh = (f_ref[:, f:f + 1] ==
                  lax.broadcasted_iota(jnp.int32, (B, 10), 1)).astype(F32)
            cols.append(oh)
        return jnp.concatenate(cols, axis=1)

    roh = onehot80(rf_ref)
    poh = onehot80(pf_ref)
    h1 = ae * jnp.dot(roh, afW_ref[...], preferred_element_type=F32)
    h2 = ae * jnp.dot(poh, afW_ref[...], preferred_element_type=F32)

    boh = (bt_ref[...] == lax.broadcasted_iota(jnp.int32, (B, NG), 1)).astype(F32)
    tn = jnp.dot(boh, t_ref[...], preferred_element_type=F32)

    node = (jnp.dot(h1, ztWh1_ref[...], preferred_element_type=F32)
            + jnp.dot(h2, ztWh2_ref[...], preferred_element_type=F32)
            + jnp.dot(tn, wtsum_ref[...], preferred_element_type=F32)
            + jnp.dot(at.astype(F32), hW0_ref[...], preferred_element_type=F32)
            + jnp.dot(roh, hWr_ref[...], preferred_element_type=F32)
            + jnp.dot(poh, hWp_ref[...], preferred_element_type=F32)
            + ztb_ref[...])
    ns_ref[...] = jnp.dot(node, W1a_ref[...], preferred_element_type=F32) + b1_ref[...]
    nd_ref[...] = jnp.dot(node, W1b_ref[...], preferred_element_type=F32)


def _edge_body(gsum_ref, gps_ref, gpd_ref, etr_ref, etp_ref,
               eW1a_ref, eW1b_ref, eb1_ref, eW2_ref, eb2_ref, bemb_ref,
               Ct_ref, Cb_ref, wd_ref, W2_ref, b2_ref, cW_ref, out_ref):
    B = gsum_ref.shape[0]
    NBT = bemb_ref.shape[0]
    dv = gps_ref[...] - gpd_ref[...]
    d = jnp.sqrt(jnp.sum(dv[:, 0:3] * dv[:, 0:3], axis=1, keepdims=True) + 1e-9)
    dT = jnp.sqrt(jnp.sum(dv[:, 3:6] * dv[:, 3:6], axis=1, keepdims=True) + 1e-9)
    e1 = jnp.maximum(d * eW1a_ref[...] + dT * eW1b_ref[...] + eb1_ref[...], 0.0)
    e_mlp = jnp.dot(e1, eW2_ref[...], preferred_element_type=F32) + eb2_ref[...]
    br = jnp.dot((etr_ref[...] == lax.broadcasted_iota(jnp.int32, (B, NBT), 1)
                  ).astype(F32), bemb_ref[...], preferred_element_type=F32)
    bp = jnp.dot((etp_ref[...] == lax.broadcasted_iota(jnp.int32, (B, NBT), 1)
                  ).astype(F32), bemb_ref[...], preferred_element_type=F32)
    pre = (gsum_ref[...]
           + jnp.dot(e_mlp * br, Ct_ref[...], preferred_element_type=F32)
           + jnp.dot(e_mlp * bp, Cb_ref[...], preferred_element_type=F32)
           + d * wd_ref[...])
    m = jnp.dot(jnp.maximum(pre, 0.0), W2_ref[...], preferred_element_type=F32) + b2_ref[...]
    phi = jnp.dot(m, cW_ref[...], preferred_element_type=F32)
    out = dv[:, 0:3] * phi
    out_ref[...] = jnp.concatenate([out, jnp.zeros((B, 5), F32)], axis=1)


def _fin_body(pos_ref, a0_ref, a1_ref, out_ref):
    out_ref[...] = pos_ref[...] + a0_ref[...] + a1_ref[...]


def kernel(atom_type, r_feat, p_feat, pos, pos_init, edge_index, edge_type_r,
           edge_type_p, t, batch, atom_emb, atom_feat_W, bond_emb, edge_W1,
           edge_b1, edge_W2, edge_b2, zt_W, zt_b, h_W, msg_W1, msg_b1,
           msg_W2, msg_b2, coord_W):
    N = atom_type.shape[0]
    E = edge_index.shape[1]
    HD = msg_W2.shape[0]
    HH = HD // 2

    at2 = atom_type[:, None].astype(jnp.int32)
    bt2 = batch[:, None].astype(jnp.int32)
    t2 = t[:, None]
    ztWh1 = zt_W[0:HH]
    ztWh2 = zt_W[HH:HD]
    hW0 = h_W[0:1]
    hWr = h_W[1:81]
    hWp = h_W[81:161]
    wtsum = zt_W[HD:HD + 1] + h_W[161:162]
    W1a = msg_W1[0:HD]
    W1b = msg_W1[HD:2 * HD]
    Ct = msg_W1[2 * HD:2 * HD + HH]
    Cb = msg_W1[2 * HD + HH:3 * HD]
    wd = msg_W1[3 * HD:3 * HD + 1]
    row1 = lambda v: v[None, :]

    NB = 1000
    n_blocks = N // NB
    full = lambda shp: pl.BlockSpec(shp, lambda i: (0, 0))
    ns, nd = pl.pallas_call(
        _node_body,
        grid=(n_blocks,),
        in_specs=[
            pl.BlockSpec((NB, 1), lambda i: (i, 0)),
            pl.BlockSpec((NB, 8), lambda i: (i, 0)),
            pl.BlockSpec((NB, 8), lambda i: (i, 0)),
            pl.BlockSpec((NB, 1), lambda i: (i, 0)),
            full(atom_emb.shape), full(atom_feat_W.shape), full(t2.shape),
            full(ztWh1.shape), full(ztWh2.shape), full(wtsum.shape),
            full((1, HD)),
            full(hW0.shape), full(hWr.shape), full(hWp.shape),
            full(W1a.shape), full(W1b.shape), full((1, HD)),
        ],
        out_specs=[pl.BlockSpec((NB, HD), lambda i: (i, 0)),
                   pl.BlockSpec((NB, HD), lambda i: (i, 0))],
        out_shape=[jax.ShapeDtypeStruct((N, HD), F32),
                   jax.ShapeDtypeStruct((N, HD), F32)],
    )(at2, r_feat.astype(jnp.int32), p_feat.astype(jnp.int32), bt2,
      atom_emb, atom_feat_W, t2, ztWh1, ztWh2, wtsum, row1(zt_b),
      hW0, hWr, hWp, W1a, W1b, row1(msg_b1))

    posc = jnp.concatenate([pos, pos_init, jnp.zeros((N, 2), F32)], axis=1)
    src = edge_index[0].astype(jnp.int32)
    dst = edge_index[1].astype(jnp.int32)

    info = plsc.get_sparse_core_info()
    NWC, NWS = info.num_cores, info.num_subcores
    NW = NWC * NWS
    EPW = E // NW
    CH = 80
    NCH = EPW // CH

    mesh = plsc.VectorSubcoreMesh(core_axis_name="c", subcore_axis_name="s")

    sc_params = pltpu.CompilerParams(use_tc_tiling_on_sc=False)

    @functools.partial(
        pl.kernel, mesh=mesh, compiler_params=sc_params,
        out_type=[jax.ShapeDtypeStruct((E, HD), F32),
                  jax.ShapeDtypeStruct((E, 8), F32),
                  jax.ShapeDtypeStruct((E, 8), F32)],
        scratch_types=[
            pltpu.VMEM((EPW,), jnp.int32), pltpu.VMEM((EPW,), jnp.int32),
            pltpu.VMEM((CH, HD), F32), pltpu.VMEM((CH, HD), F32),
            pltpu.VMEM((CH, HD), F32),
            pltpu.VMEM((CH, HD), F32), pltpu.VMEM((CH, HD), F32),
            pltpu.VMEM((CH, HD), F32),
            pltpu.VMEM((CH, 8), F32), pltpu.VMEM((CH, 8), F32),
            pltpu.VMEM((CH, 8), F32),
            pltpu.VMEM((CH, 8), F32), pltpu.VMEM((CH, 8), F32),
            pltpu.VMEM((CH, 8), F32),
            pltpu.SemaphoreType.DMA, pltpu.SemaphoreType.DMA,
            pltpu.SemaphoreType.DMA, pltpu.SemaphoreType.DMA,
            pltpu.SemaphoreType.DMA, pltpu.SemaphoreType.DMA,
        ],
    )
    def _gather_k(src_h, dst_h, ns_h, nd_h, posc_h,
                  gs_h, ps_h, pd_h,
                  idxs_v, idxd_v, rs0, rs1, rs2, rd0, rd1, rd2,
                  pvs0, pvs1, pvs2, pvd0, pvd1, pvd2,
                  gsem0, gsem1, gsem2, wsem0, wsem1, wsem2):
        wid = lax.axis_index("s") * NWC + lax.axis_index("c")
        base = wid * EPW
        bufs = ((rs0, rd0, pvs0, pvd0), (rs1, rd1, pvs1, pvd1),
                (rs2, rd2, pvs2, pvd2))
        gsems = (gsem0, gsem1, gsem2)
        wsems = (wsem0, wsem1, wsem2)
        pltpu.sync_copy(src_h.at[pl.ds(base, EPW)], idxs_v)
        pltpu.sync_copy(dst_h.at[pl.ds(base, EPW)], idxd_v)

        def fire_gathers(g, b):
            rs, rd, pvs, pvd = bufs[b]
            isl = idxs_v.at[pl.ds(g * CH, CH)]
            dsl = idxd_v.at[pl.ds(g * CH, CH)]
            pltpu.async_copy(ns_h.at[isl], rs, gsems[b])
            pltpu.async_copy(nd_h.at[dsl], rd, gsems[b])
            pltpu.async_copy(posc_h.at[isl], pvs, gsems[b])
            pltpu.async_copy(posc_h.at[dsl], pvd, gsems[b])

        def drain_gathers(b):
            rs, rd, pvs, pvd = bufs[b]
            pltpu.make_async_copy(ns_h.at[pl.ds(0, CH)], rs, gsems[b]).wait()
            pltpu.make_async_copy(nd_h.at[pl.ds(0, CH)], rd, gsems[b]).wait()
            pltpu.make_async_copy(posc_h.at[pl.ds(0, CH)], pvs, gsems[b]).wait()
            pltpu.make_async_copy(posc_h.at[pl.ds(0, CH)], pvd, gsems[b]).wait()

        def sum_rows(b):
            rs, rd, _, _ = bufs[b]

            def add_row(r, carry):
                for c in range(HD // 16):
                    sl = pl.ds(c * 16, 16)
                    rs[r, sl] = rs[r, sl] + rd[r, sl]
                return carry

            lax.fori_loop(0, CH, add_row, 0)

        def fire_writes(g, b):
            rs, rd, pvs, pvd = bufs[b]
            off = base + g * CH
            pltpu.async_copy(rs, gs_h.at[pl.ds(off, CH)], wsems[b])
            pltpu.async_copy(pvs, ps_h.at[pl.ds(off, CH)], wsems[b])
            pltpu.async_copy(pvd, pd_h.at[pl.ds(off, CH)], wsems[b])

        def drain_writes(b):
            rs, rd, pvs, pvd = bufs[b]
            pltpu.make_async_copy(rs, gs_h.at[pl.ds(0, CH)], wsems[b]).wait()
            pltpu.make_async_copy(pvs, ps_h.at[pl.ds(0, CH)], wsems[b]).wait()
            pltpu.make_async_copy(pvd, pd_h.at[pl.ds(0, CH)], wsems[b]).wait()

        fire_gathers(0, 0)
        fire_gathers(1, 1)

        def step(g, b):
            nb = (b + 2) % 3

            @pl.when(g + 2 < NCH)
            def _():
                @pl.when(g >= 1)
                def _():
                    drain_writes(nb)
                fire_gathers(g + 2, nb)
            drain_gathers(b)
            sum_rows(b)
            fire_writes(g, b)

        def body(g, carry):
            @pl.when(g % 3 == 0)
            def _():
                step(g, 0)

            @pl.when(g % 3 == 1)
            def _():
                step(g, 1)

            @pl.when(g % 3 == 2)
            def _():
                step(g, 2)

            return carry

        lax.fori_loop(0, NCH, body, 0)
        drain_writes((NCH - 3) % 3)
        drain_writes((NCH - 2) % 3)
        drain_writes((NCH - 1) % 3)

    gsum, gps, gpd = _gather_k(src, dst, ns, nd, posc)

    EB = 6400
    e_blocks = E // EB
    edgeout = pl.pallas_call(
        _edge_body,
        grid=(e_blocks,),
        in_specs=[
            pl.BlockSpec((EB, HD), lambda i: (i, 0)),
            pl.BlockSpec((EB, 8), lambda i: (i, 0)),
            pl.BlockSpec((EB, 8), lambda i: (i, 0)),
            pl.BlockSpec((EB, 1), lambda i: (i, 0)),
            pl.BlockSpec((EB, 1), lambda i: (i, 0)),
            full((1, HH)), full((1, HH)), full((1, HH)), full(edge_W2.shape),
            full((1, HH)), full(bond_emb.shape),
            full(Ct.shape), full(Cb.shape), full(wd.shape),
            full(msg_W2.shape), full((1, HD)), full(coord_W.shape),
        ],
        out_specs=pl.BlockSpec((EB, 8), lambda i: (i, 0)),
        out_shape=jax.ShapeDtypeStruct((E, 8), F32),
    )(gsum, gps, gpd, edge_type_r[:, None].astype(jnp.int32),
      edge_type_p[:, None].astype(jnp.int32),
      edge_W1[0:1], edge_W1[1:2], row1(edge_b1), edge_W2, row1(edge_b2),
      bond_emb, Ct, Cb, wd, msg_W2, row1(msg_b2), coord_W)

    NPAD = 10240
    RPT = NPAD // NWS
    zeros_acc = jnp.zeros((NPAD, 8), F32)

    @functools.partial(
        pl.kernel, mesh=mesh, compiler_params=sc_params,
        out_type=[jax.ShapeDtypeStruct((2 * NPAD, 8), F32)],
        scratch_types=[
            pltpu.VMEM((CH,), jnp.int32),
            pltpu.VMEM((CH, 8), F32),
            pltpu.VMEM((RPT, 8), F32),
            pltpu.VMEM_SHARED((NPAD, 8), F32),
        ],
    )
    def _scatter_k(dst_h, eo_h, z_h, agg_h, idx_v, rows_v, tmp_v, acc_s):
        cid = lax.axis_index("c")
        sid = lax.axis_index("s")
        wid = sid * NWC + cid
        base = wid * EPW
        pltpu.sync_copy(z_h.at[pl.ds(sid * RPT, RPT)], tmp_v)
        pltpu.sync_copy(tmp_v, acc_s.at[pl.ds(sid * RPT, RPT)])
        plsc.subcore_barrier()

        def body(g, carry):
            off = base + g * CH
            pltpu.sync_copy(dst_h.at[pl.ds(off, CH)], idx_v)
            pltpu.sync_copy(eo_h.at[pl.ds(off, CH)], rows_v)
            pltpu.sync_copy(rows_v, acc_s.at[idx_v], add=True)
            return carry

        lax.fori_loop(0, NCH, body, 0)
        plsc.subcore_barrier()
        pltpu.sync_copy(acc_s.at[pl.ds(sid * RPT, RPT)], tmp_v)
        pltpu.sync_copy(tmp_v, agg_h.at[pl.ds(cid * NPAD + sid * RPT, RPT)])

    (agg,) = _scatter_k(dst, edgeout, zeros_acc)

    a0 = agg[0:N, 0:3]
    a1 = agg[NPAD:NPAD + N, 0:3]
    FB = 1000
    pred = pl.pallas_call(
        _fin_body,
        grid=(N // FB,),
        in_specs=[pl.BlockSpec((FB, 3), lambda i: (i, 0))] * 3,
        out_specs=pl.BlockSpec((FB, 3), lambda i: (i, 0)),
        out_shape=jax.ShapeDtypeStruct((N, 3), F32),
    )(pos, a0, a1)
    return pred
